# baseline (speedup 1.0000x reference)
import numpy as np
import jax
import jax.numpy as jnp
from jax.experimental import pallas as pl
from jax.experimental.pallas import tpu as pltpu

E = 8
C = 96
HID = 256
B = 8
H = 56
W = 56
N = H * W
NP = 58 * 56
EC = E * C
KC = 384
KK = 3 * KC
G = 8
CG = C // G
MT = 392
NSUB = N // MT
EPS = 1e-5
CNT = float(N * CG)


def _pce_body(f3_ref, f3prev_ref, wcol_ref, brow_ref, gnw_ref, gnb_ref,
              rw1_ref, rb1_ref, rw2_ref, rb2_ref, mw_ref, mb_ref,
              m768_ref, m64e_ref, m96_ref, m8e_ref, msel_ref, mexp_ref,
              out_ref, y_scr, acc_scr, stat_scr, s_scr):
    sid = pl.program_id(0)
    par = jax.lax.rem(sid, 2)
    oar = 1 - par
    wcol = wcol_ref[...]
    ones_mt = jnp.ones((1, MT), jnp.float32)

    A_pv = stat_scr[pl.ds((oar * 5 + 0) * 8, 1), :]
    B_pv = stat_scr[pl.ds((oar * 5 + 1) * 8, 1), :]
    wts_pv = stat_scr[pl.ds((oar * 5 + 2) * 8, 1), 0:E]
    sw_pv = jnp.sum(wts_pv, axis=-1, keepdims=True)
    S_pv = s_scr[pl.ds(oar * EC, EC), :]
    A2_pv = stat_scr[pl.ds((par * 5 + 3) * 8, 1), 0:C]
    B2_pv = stat_scr[pl.ds((par * 5 + 4) * 8, 1), 0:C]

    s_acc = jnp.zeros((1, EC), jnp.float32)
    q_acc = jnp.zeros((1, EC), jnp.float32)
    g_acc = jnp.zeros((1, C), jnp.float32)
    ms = jnp.zeros((1, C), jnp.float32)
    mq = jnp.zeros((1, C), jnp.float32)

    for i in range(NSUB):
        r0 = i * MT
        xc = jnp.concatenate(
            [f3_ref[0, 56 * ky + r0:56 * ky + r0 + MT, :] for ky in range(3)],
            axis=-1)
        yt = jnp.dot(xc, wcol, preferred_element_type=jnp.float32)
        y_scr[pl.ds(par * N + r0, MT), :] = yt.astype(jnp.bfloat16)
        s_acc = s_acc + jnp.dot(ones_mt, yt,
                                preferred_element_type=jnp.float32)
        q_acc = q_acc + jnp.dot(ones_mt, yt * yt,
                                preferred_element_type=jnp.float32)
        g_acc = g_acc + jnp.dot(
            ones_mt, f3_ref[0, 56 + r0:56 + r0 + MT, C:2 * C]
            .astype(jnp.float32), preferred_element_type=jnp.float32)
        ytp = y_scr[pl.ds(oar * N + r0, MT), :].astype(jnp.float32)
        act = jnp.maximum(ytp * A_pv + B_pv, 0.0).astype(jnp.bfloat16)
        acc = jnp.dot(act, S_pv, preferred_element_type=jnp.float32)
        xres = f3prev_ref[0, 56 + r0:56 + r0 + MT, C:2 * C]
        acc = acc + xres.astype(jnp.float32) * sw_pv
        acc_scr[pl.ds(oar * N + r0, MT), :] = acc
        ms = ms + jnp.dot(ones_mt, acc, preferred_element_type=jnp.float32)
        mq = mq + jnp.dot(ones_mt, acc * acc,
                          preferred_element_type=jnp.float32)
        out_ref[0, r0:r0 + MT, :] = (
            acc_scr[pl.ds(par * N + r0, MT), :] * A2_pv + B2_pv)

    brow = brow_ref[...]
    s2 = s_acc + N * brow
    q2 = q_acc + 2.0 * brow * s_acc + N * brow * brow
    gs = jnp.dot(s2, m768_ref[...])
    gq = jnp.dot(q2, m768_ref[...])
    mu = gs / CNT
    var = gq / CNT - mu * mu
    inv = jax.lax.rsqrt(var + EPS)
    mu_c = jnp.dot(mu, m64e_ref[...])
    inv_c = jnp.dot(inv, m64e_ref[...])
    gnw = gnw_ref[...]
    A = inv_c * gnw
    Bc = (brow - mu_c) * inv_c * gnw + gnb_ref[...]

    g = g_acc / float(N)
    h1 = jnp.maximum(jnp.dot(g, rw1_ref[...]) + rb1_ref[...], 0.0)
    lg = jnp.dot(h1, rw2_ref[...]) + rb2_ref[...]
    lg = lg - jnp.max(lg, axis=-1, keepdims=True)
    ew = jnp.exp(lg)
    wts = ew / jnp.sum(ew, axis=-1, keepdims=True)
    wcolv = jnp.dot(mexp_ref[...], jnp.transpose(wts))
    S = (msel_ref[...] * wcolv).astype(jnp.bfloat16)

    stat_scr[pl.ds((par * 5 + 0) * 8, 1), :] = A
    stat_scr[pl.ds((par * 5 + 1) * 8, 1), :] = Bc
    stat_scr[pl.ds((par * 5 + 2) * 8, 1), 0:E] = wts
    s_scr[pl.ds(par * EC, EC), :] = S

    gs2 = jnp.dot(ms, m96_ref[...])
    gq2 = jnp.dot(mq, m96_ref[...])
    mu2 = gs2 / CNT
    var2 = gq2 / CNT - mu2 * mu2
    inv2 = jax.lax.rsqrt(var2 + EPS)
    mu2_c = jnp.dot(mu2, m8e_ref[...])
    inv2_c = jnp.dot(inv2, m8e_ref[...])
    A2 = inv2_c * mw_ref[...]
    B2 = mb_ref[...] - mu2_c * A2
    stat_scr[pl.ds((oar * 5 + 3) * 8, 1), 0:C] = A2
    stat_scr[pl.ds((oar * 5 + 4) * 8, 1), 0:C] = B2


def kernel(x, Wexp, bexp, gn_w, gn_b, rW1, rb1, rW2, rb2, merge_w, merge_b):
    xt = jnp.transpose(x, (0, 2, 3, 1))
    xp = jnp.pad(xt, ((0, 0), (1, 1), (1, 1), (0, 0)))
    f3 = jnp.concatenate([xp[:, :, k:k + W, :] for k in range(3)],
                         axis=-1)
    f3 = jnp.pad(f3, ((0, 0), (0, 0), (0, 0), (0, KC - 3 * C)))
    f3 = f3.reshape(B, NP, KC).astype(jnp.bfloat16)
    wc = jnp.transpose(Wexp, (3, 4, 2, 0, 1)).reshape(3, 3 * C, EC)
    wc = jnp.pad(wc, ((0, 0), (0, KC - 3 * C), (0, 0))).reshape(KK, EC)
    wc = wc.astype(jnp.bfloat16)

    brow = bexp.reshape(1, EC)
    gnw_row = gn_w.reshape(1, EC)
    gnb_row = gn_b.reshape(1, EC)
    rb1_row = rb1.reshape(1, HID)
    rb2_row = rb2.reshape(1, E)
    mw_row = merge_w.reshape(1, C)
    mb_row = merge_b.reshape(1, C)

    cidx = np.arange(EC)
    gidx = (cidx // C) * G + (cidx % C) // CG
    m768 = (gidx[:, None] == np.arange(E * G)[None, :]).astype(np.float32)
    m64e = m768.T.copy()
    c96 = np.arange(C)
    m96 = ((c96 // CG)[:, None] == np.arange(G)[None, :]).astype(np.float32)
    m8e = m96.T.copy()
    msel = ((cidx % C)[:, None] == c96[None, :]).astype(np.float32)
    mexp = ((cidx // C)[:, None] == np.arange(E)[None, :]).astype(np.float32)

    const = lambda s: (0, 0)
    out = pl.pallas_call(
        _pce_body,
        grid=(B + 2,),
        in_specs=[
            pl.BlockSpec((1, NP, KC), lambda s: (jnp.minimum(s, B - 1), 0, 0)),
            pl.BlockSpec((1, NP, KC),
                         lambda s: (jnp.clip(s - 1, 0, B - 1), 0, 0)),
            pl.BlockSpec((KK, EC), const),
            pl.BlockSpec((1, EC), const),
            pl.BlockSpec((1, EC), const),
            pl.BlockSpec((1, EC), const),
            pl.BlockSpec((C, HID), const),
            pl.BlockSpec((1, HID), const),
            pl.BlockSpec((HID, E), const),
            pl.BlockSpec((1, E), const),
            pl.BlockSpec((1, C), const),
            pl.BlockSpec((1, C), const),
            pl.BlockSpec((EC, E * G), const),
            pl.BlockSpec((E * G, EC), const),
            pl.BlockSpec((C, G), const),
            pl.BlockSpec((G, C), const),
            pl.BlockSpec((EC, C), const),
            pl.BlockSpec((EC, E), const),
        ],
        out_specs=pl.BlockSpec((1, N, C),
                               lambda s: (jnp.maximum(s - 2, 0), 0, 0)),
        out_shape=jax.ShapeDtypeStruct((B, N, C), jnp.float32),
        scratch_shapes=[
            pltpu.VMEM((2 * N, EC), jnp.bfloat16),
            pltpu.VMEM((2 * N, C), jnp.float32),
            pltpu.VMEM((80, EC), jnp.float32),
            pltpu.VMEM((2 * EC, C), jnp.bfloat16),
        ],
    )(f3, f3, wc, brow, gnw_row, gnb_row, rW1, rb1_row, rW2, rb2_row,
      mw_row, mb_row, jnp.asarray(m768), jnp.asarray(m64e),
      jnp.asarray(m96), jnp.asarray(m8e), jnp.asarray(msel),
      jnp.asarray(mexp))

    return jnp.transpose(out.reshape(B, H, W, C), (0, 3, 1, 2))


# speedup vs baseline: 1.0402x; 1.0402x over previous
import numpy as np
import jax
import jax.numpy as jnp
from jax.experimental import pallas as pl
from jax.experimental.pallas import tpu as pltpu

E = 8
C = 96
HID = 256
B = 8
H = 56
W = 56
N = H * W
NP = 58 * 56
EC = E * C
KC = 384
KK = 3 * KC
G = 8
CG = C // G
MT = 448
NSUB = N // MT
EPS = 1e-5
CNT = float(N * CG)


def _pce_body(f3_ref, f3prev_ref, wcol_ref, brow_ref, gnw_ref, gnb_ref,
              rw1_ref, rb1_ref, rw2_ref, rb2_ref, mw_ref, mb_ref,
              m768_ref, m64e_ref, m96_ref, m8e_ref, msel_ref, mexp_ref,
              out_ref, y_scr, acc_scr, stat_scr, s_scr):
    sid = pl.program_id(0)
    par = jax.lax.rem(sid, 2)
    oar = 1 - par
    wcol = wcol_ref[...]
    ones_mt = jnp.ones((1, MT), jnp.float32)

    A_pv = stat_scr[pl.ds((oar * 5 + 0) * 8, 1), :]
    B_pv = stat_scr[pl.ds((oar * 5 + 1) * 8, 1), :]
    wts_pv = stat_scr[pl.ds((oar * 5 + 2) * 8, 1), 0:E]
    sw_pv = jnp.sum(wts_pv, axis=-1, keepdims=True)
    S_pv = s_scr[pl.ds(oar * EC, EC), :]
    A2_pv = stat_scr[pl.ds((par * 5 + 3) * 8, 1), 0:C]
    B2_pv = stat_scr[pl.ds((par * 5 + 4) * 8, 1), 0:C]

    s_acc = jnp.zeros((1, EC), jnp.float32)
    q_acc = jnp.zeros((1, EC), jnp.float32)
    g_acc = jnp.zeros((1, C), jnp.float32)
    ms = jnp.zeros((1, C), jnp.float32)
    mq = jnp.zeros((1, C), jnp.float32)

    for i in range(NSUB):
        r0 = i * MT
        xc = jnp.concatenate(
            [f3_ref[0, 56 * ky + r0:56 * ky + r0 + MT, :] for ky in range(3)],
            axis=-1)
        yt = jnp.dot(xc, wcol, preferred_element_type=jnp.float32)
        y_scr[pl.ds(par * N + r0, MT), :] = yt.astype(jnp.bfloat16)
        s_acc = s_acc + jnp.dot(ones_mt, yt,
                                preferred_element_type=jnp.float32)
        q_acc = q_acc + jnp.dot(ones_mt, yt * yt,
                                preferred_element_type=jnp.float32)
        g_acc = g_acc + jnp.dot(
            ones_mt, f3_ref[0, 56 + r0:56 + r0 + MT, C:2 * C]
            .astype(jnp.float32), preferred_element_type=jnp.float32)
        ytp = y_scr[pl.ds(oar * N + r0, MT), :].astype(jnp.float32)
        act = jnp.maximum(ytp * A_pv + B_pv, 0.0).astype(jnp.bfloat16)
        acc = jnp.dot(act, S_pv, preferred_element_type=jnp.float32)
        xres = f3prev_ref[0, 56 + r0:56 + r0 + MT, C:2 * C]
        acc = acc + xres.astype(jnp.float32) * sw_pv
        acc_scr[pl.ds(oar * N + r0, MT), :] = acc
        ms = ms + jnp.dot(ones_mt, acc, preferred_element_type=jnp.float32)
        mq = mq + jnp.dot(ones_mt, acc * acc,
                          preferred_element_type=jnp.float32)
        out_ref[0, r0:r0 + MT, :] = (
            acc_scr[pl.ds(par * N + r0, MT), :] * A2_pv + B2_pv)

    brow = brow_ref[...]
    s2 = s_acc + N * brow
    q2 = q_acc + 2.0 * brow * s_acc + N * brow * brow
    gs = jnp.dot(s2, m768_ref[...])
    gq = jnp.dot(q2, m768_ref[...])
    mu = gs / CNT
    var = gq / CNT - mu * mu
    inv = jax.lax.rsqrt(var + EPS)
    mu_c = jnp.dot(mu, m64e_ref[...])
    inv_c = jnp.dot(inv, m64e_ref[...])
    gnw = gnw_ref[...]
    A = inv_c * gnw
    Bc = (brow - mu_c) * inv_c * gnw + gnb_ref[...]

    g = g_acc / float(N)
    h1 = jnp.maximum(jnp.dot(g, rw1_ref[...]) + rb1_ref[...], 0.0)
    lg = jnp.dot(h1, rw2_ref[...]) + rb2_ref[...]
    lg = lg - jnp.max(lg, axis=-1, keepdims=True)
    ew = jnp.exp(lg)
    wts = ew / jnp.sum(ew, axis=-1, keepdims=True)
    wcolv = jnp.dot(mexp_ref[...], jnp.transpose(wts))
    S = (msel_ref[...] * wcolv).astype(jnp.bfloat16)

    stat_scr[pl.ds((par * 5 + 0) * 8, 1), :] = A
    stat_scr[pl.ds((par * 5 + 1) * 8, 1), :] = Bc
    stat_scr[pl.ds((par * 5 + 2) * 8, 1), 0:E] = wts
    s_scr[pl.ds(par * EC, EC), :] = S

    gs2 = jnp.dot(ms, m96_ref[...])
    gq2 = jnp.dot(mq, m96_ref[...])
    mu2 = gs2 / CNT
    var2 = gq2 / CNT - mu2 * mu2
    inv2 = jax.lax.rsqrt(var2 + EPS)
    mu2_c = jnp.dot(mu2, m8e_ref[...])
    inv2_c = jnp.dot(inv2, m8e_ref[...])
    A2 = inv2_c * mw_ref[...]
    B2 = mb_ref[...] - mu2_c * A2
    stat_scr[pl.ds((oar * 5 + 3) * 8, 1), 0:C] = A2
    stat_scr[pl.ds((oar * 5 + 4) * 8, 1), 0:C] = B2


def kernel(x, Wexp, bexp, gn_w, gn_b, rW1, rb1, rW2, rb2, merge_w, merge_b):
    xt = jnp.transpose(x, (0, 2, 3, 1))
    xp = jnp.pad(xt, ((0, 0), (1, 1), (1, 1), (0, 0)))
    f3 = jnp.concatenate([xp[:, :, k:k + W, :] for k in range(3)],
                         axis=-1)
    f3 = jnp.pad(f3, ((0, 0), (0, 0), (0, 0), (0, KC - 3 * C)))
    f3 = f3.reshape(B, NP, KC).astype(jnp.bfloat16)
    wc = jnp.transpose(Wexp, (3, 4, 2, 0, 1)).reshape(3, 3 * C, EC)
    wc = jnp.pad(wc, ((0, 0), (0, KC - 3 * C), (0, 0))).reshape(KK, EC)
    wc = wc.astype(jnp.bfloat16)

    brow = bexp.reshape(1, EC)
    gnw_row = gn_w.reshape(1, EC)
    gnb_row = gn_b.reshape(1, EC)
    rb1_row = rb1.reshape(1, HID)
    rb2_row = rb2.reshape(1, E)
    mw_row = merge_w.reshape(1, C)
    mb_row = merge_b.reshape(1, C)

    cidx = np.arange(EC)
    gidx = (cidx // C) * G + (cidx % C) // CG
    m768 = (gidx[:, None] == np.arange(E * G)[None, :]).astype(np.float32)
    m64e = m768.T.copy()
    c96 = np.arange(C)
    m96 = ((c96 // CG)[:, None] == np.arange(G)[None, :]).astype(np.float32)
    m8e = m96.T.copy()
    msel = ((cidx % C)[:, None] == c96[None, :]).astype(np.float32)
    mexp = ((cidx // C)[:, None] == np.arange(E)[None, :]).astype(np.float32)

    const = lambda s: (0, 0)
    out = pl.pallas_call(
        _pce_body,
        grid=(B + 2,),
        in_specs=[
            pl.BlockSpec((1, NP, KC), lambda s: (jnp.minimum(s, B - 1), 0, 0)),
            pl.BlockSpec((1, NP, KC),
                         lambda s: (jnp.clip(s - 1, 0, B - 1), 0, 0)),
            pl.BlockSpec((KK, EC), const),
            pl.BlockSpec((1, EC), const),
            pl.BlockSpec((1, EC), const),
            pl.BlockSpec((1, EC), const),
            pl.BlockSpec((C, HID), const),
            pl.BlockSpec((1, HID), const),
            pl.BlockSpec((HID, E), const),
            pl.BlockSpec((1, E), const),
            pl.BlockSpec((1, C), const),
            pl.BlockSpec((1, C), const),
            pl.BlockSpec((EC, E * G), const),
            pl.BlockSpec((E * G, EC), const),
            pl.BlockSpec((C, G), const),
            pl.BlockSpec((G, C), const),
            pl.BlockSpec((EC, C), const),
            pl.BlockSpec((EC, E), const),
        ],
        out_specs=pl.BlockSpec((1, N, C),
                               lambda s: (jnp.maximum(s - 2, 0), 0, 0)),
        out_shape=jax.ShapeDtypeStruct((B, N, C), jnp.float32),
        scratch_shapes=[
            pltpu.VMEM((2 * N, EC), jnp.bfloat16),
            pltpu.VMEM((2 * N, C), jnp.float32),
            pltpu.VMEM((80, EC), jnp.float32),
            pltpu.VMEM((2 * EC, C), jnp.bfloat16),
        ],
    )(f3, f3, wc, brow, gnw_row, gnb_row, rW1, rb1_row, rW2, rb2_row,
      mw_row, mb_row, jnp.asarray(m768), jnp.asarray(m64e),
      jnp.asarray(m96), jnp.asarray(m8e), jnp.asarray(msel),
      jnp.asarray(mexp))

    return jnp.transpose(out.reshape(B, H, W, C), (0, 3, 1, 2))
